# baseline (speedup 1.0000x reference)
import functools

import jax
import jax.numpy as jnp
from jax import lax
from jax.experimental import pallas as pl
from jax.experimental.pallas import tpu as pltpu
from jax.experimental.pallas import tpu_sc as plsc

N = 10000
D_IN = 256
D_HID = 256
D_OUT = 4
N_EDGES = 160000

NC = 2
NS = 16
CH = 128

RPT = N // NS
EROWS = N_EDGES // CH
C1 = EROWS // NS
R1 = EROWS % NS
C2 = EROWS // (NC * NS)
R2 = EROWS % (NC * NS)
NW = 16

_mesh = plsc.VectorSubcoreMesh(
    core_axis_name="c", subcore_axis_name="s", num_cores=NC, num_subcores=NS)

_sc_params = pltpu.CompilerParams(use_tc_tiling_on_sc=False)

f32 = jnp.float32
bf16 = jnp.bfloat16


def _ring(n, wait_gather, fire_scatter, wait_scatter, fire_gather):
    fire_gather(0, 0)
    fire_gather(1, 1)

    def body(i, carry):
        for b in range(4):
            j = i * 4 + b

            @pl.when(j < n)
            def _():
                wait_gather(j, b)
                fire_scatter(j, b)

            bn = (b + 2) % 4

            @pl.when((j >= 2) & (j <= n + 1))
            def _():
                wait_scatter(j - 2, bn)

            @pl.when(j + 2 < n)
            def _():
                fire_gather(j + 2, bn)
        return carry

    lax.fori_loop(0, lax.div(n + 5, 4), body, 0)


@functools.partial(
    pl.kernel,
    out_type=(jax.ShapeDtypeStruct((N, NW), f32),
              jax.ShapeDtypeStruct((N, NW), f32)),
    mesh=_mesh,
    compiler_params=_sc_params,
    scratch_types=[pltpu.VMEM((C2 + 1, CH), jnp.int32),
                   pltpu.VMEM((CH, NW), f32),
                   pltpu.VMEM_SHARED((N, NW), f32),
                   pltpu.SemaphoreType.DMA],
)
def _deg_kernel(ei, zeros_hbm, ones_hbm, deg0, deg1, idx_v, ones_v, acc_sh,
                sem):
    c = lax.axis_index("c")
    s = lax.axis_index("s")
    rowbase = s * RPT
    pltpu.sync_copy(ones_hbm, ones_v)
    pltpu.sync_copy(zeros_hbm.at[pl.ds(rowbase, RPT)],
                    acc_sh.at[pl.ds(rowbase, RPT)])
    wid = c * NS + s
    nrow = C2 + (wid < R2)
    erow = wid * C2 + jnp.minimum(wid, R2)
    pltpu.sync_copy(ei.at[1, pl.ds(erow, C2)], idx_v.at[pl.ds(0, C2)])

    @pl.when(wid < R2)
    def _():
        pltpu.sync_copy(ei.at[1, pl.ds(erow + C2, 1)], idx_v.at[pl.ds(C2, 1)])

    plsc.subcore_barrier()

    def body(j, carry):
        pltpu.async_copy(ones_v, acc_sh.at[idx_v.at[j]], sem, add=True)
        return carry

    lax.fori_loop(0, nrow, body, 0)

    def drain(j, carry):
        pltpu.make_async_copy(ones_v, acc_sh.at[idx_v.at[j]], sem).wait()
        return carry

    lax.fori_loop(0, nrow, drain, 0)
    plsc.subcore_barrier()

    @pl.when(c == 0)
    def _():
        pltpu.sync_copy(acc_sh.at[pl.ds(rowbase, RPT)],
                        deg0.at[pl.ds(rowbase, RPT)])

    @pl.when(c == 1)
    def _():
        pltpu.sync_copy(acc_sh.at[pl.ds(rowbase, RPT)],
                        deg1.at[pl.ds(rowbase, RPT)])


@functools.partial(
    pl.kernel,
    out_type=(jax.ShapeDtypeStruct((N, 128), bf16),
              jax.ShapeDtypeStruct((N, 128), bf16)),
    mesh=_mesh,
    compiler_params=_sc_params,
    scratch_types=[pltpu.VMEM((C1 + 1, CH), jnp.int32),
                   pltpu.VMEM((C1 + 1, CH), jnp.int32),
                   pltpu.VMEM((4 * CH, 128), bf16),
                   pltpu.VMEM_SHARED((N, 128), bf16),
                   pltpu.SemaphoreType.DMA, pltpu.SemaphoreType.DMA,
                   pltpu.SemaphoreType.DMA, pltpu.SemaphoreType.DMA,
                   pltpu.SemaphoreType.DMA, pltpu.SemaphoreType.DMA,
                   pltpu.SemaphoreType.DMA, pltpu.SemaphoreType.DMA],
)
def _agg1_kernel(ylo, yhi, ei, alo, ahi,
                 src_v, dst_v, rows_v, acc_sh,
                 g0, g1, g2, g3, s0, s1, s2, s3):
    gsems = (g0, g1, g2, g3)
    ssems = (s0, s1, s2, s3)
    c = lax.axis_index("c")
    s = lax.axis_index("s")
    rowbase = s * RPT
    nrow = C1 + (s < R1)
    erow = s * C1 + jnp.minimum(s, R1)
    pltpu.sync_copy(ei.at[0, pl.ds(erow, C1)], src_v.at[pl.ds(0, C1)])
    pltpu.sync_copy(ei.at[1, pl.ds(erow, C1)], dst_v.at[pl.ds(0, C1)])

    @pl.when(s < R1)
    def _():
        pltpu.sync_copy(ei.at[0, pl.ds(erow + C1, 1)], src_v.at[pl.ds(C1, 1)])
        pltpu.sync_copy(ei.at[1, pl.ds(erow + C1, 1)], dst_v.at[pl.ds(C1, 1)])

    @pl.when(c == 0)
    def _():
        pltpu.sync_copy(ylo.at[pl.ds(rowbase, RPT)],
                        acc_sh.at[pl.ds(rowbase, RPT)])

    @pl.when(c == 1)
    def _():
        pltpu.sync_copy(yhi.at[pl.ds(rowbase, RPT)],
                        acc_sh.at[pl.ds(rowbase, RPT)])

    plsc.subcore_barrier()

    def buf(b):
        return rows_v.at[pl.ds(b * CH, CH)]

    def fire_gather(jn, b):
        @pl.when(c == 0)
        def _():
            pltpu.async_copy(ylo.at[src_v.at[jn]], buf(b), gsems[b])

        @pl.when(c == 1)
        def _():
            pltpu.async_copy(yhi.at[src_v.at[jn]], buf(b), gsems[b])

    def wait_gather(j, b):
        pltpu.make_async_copy(ylo.at[src_v.at[j]], buf(b), gsems[b]).wait()

    def fire_scatter(j, b):
        pltpu.async_copy(buf(b), acc_sh.at[dst_v.at[j]], ssems[b], add=True)

    def wait_scatter(j, b):
        pltpu.make_async_copy(buf(b), acc_sh.at[dst_v.at[j]],
                              ssems[b]).wait()

    _ring(nrow, wait_gather, fire_scatter, wait_scatter, fire_gather)
    plsc.subcore_barrier()

    @pl.when(c == 0)
    def _():
        pltpu.sync_copy(acc_sh.at[pl.ds(rowbase, RPT)],
                        alo.at[pl.ds(rowbase, RPT)])

    @pl.when(c == 1)
    def _():
        pltpu.sync_copy(acc_sh.at[pl.ds(rowbase, RPT)],
                        ahi.at[pl.ds(rowbase, RPT)])


@functools.partial(
    pl.kernel,
    out_type=(jax.ShapeDtypeStruct((N, NW), f32),
              jax.ShapeDtypeStruct((N, NW), f32)),
    mesh=_mesh,
    compiler_params=_sc_params,
    scratch_types=[pltpu.VMEM((C2 + 1, CH), jnp.int32),
                   pltpu.VMEM((C2 + 1, CH), jnp.int32),
                   pltpu.VMEM((4 * CH, NW), f32),
                   pltpu.VMEM_SHARED((N, NW), f32),
                   pltpu.SemaphoreType.DMA, pltpu.SemaphoreType.DMA,
                   pltpu.SemaphoreType.DMA, pltpu.SemaphoreType.DMA,
                   pltpu.SemaphoreType.DMA, pltpu.SemaphoreType.DMA,
                   pltpu.SemaphoreType.DMA, pltpu.SemaphoreType.DMA],
)
def _agg2_kernel(y2, ei, a0, a1, src_v, dst_v, rows_v, acc_sh,
                 g0, g1, g2, g3, s0, s1, s2, s3):
    gsems = (g0, g1, g2, g3)
    ssems = (s0, s1, s2, s3)
    c = lax.axis_index("c")
    s = lax.axis_index("s")
    rowbase = s * RPT
    wid = c * NS + s
    nrow = C2 + (wid < R2)
    erow = wid * C2 + jnp.minimum(wid, R2)
    pltpu.sync_copy(ei.at[0, pl.ds(erow, C2)], src_v.at[pl.ds(0, C2)])
    pltpu.sync_copy(ei.at[1, pl.ds(erow, C2)], dst_v.at[pl.ds(0, C2)])

    @pl.when(wid < R2)
    def _():
        pltpu.sync_copy(ei.at[0, pl.ds(erow + C2, 1)], src_v.at[pl.ds(C2, 1)])
        pltpu.sync_copy(ei.at[1, pl.ds(erow + C2, 1)], dst_v.at[pl.ds(C2, 1)])
    pltpu.sync_copy(y2.at[pl.ds(rowbase, RPT)],
                    acc_sh.at[pl.ds(rowbase, RPT)])
    plsc.subcore_barrier()

    def buf(b):
        return rows_v.at[pl.ds(b * CH, CH)]

    def fire_gather(jn, b):
        pltpu.async_copy(y2.at[src_v.at[jn]], buf(b), gsems[b])

    def wait_gather(j, b):
        pltpu.make_async_copy(y2.at[src_v.at[j]], buf(b), gsems[b]).wait()

    def fire_scatter(j, b):
        pltpu.async_copy(buf(b), acc_sh.at[dst_v.at[j]], ssems[b], add=True)

    def wait_scatter(j, b):
        pltpu.make_async_copy(buf(b), acc_sh.at[dst_v.at[j]],
                              ssems[b]).wait()

    _ring(nrow, wait_gather, fire_scatter, wait_scatter, fire_gather)
    plsc.subcore_barrier()

    @pl.when(c == 0)
    def _():
        pltpu.sync_copy(acc_sh.at[pl.ds(rowbase, RPT)],
                        a0.at[pl.ds(rowbase, RPT)])

    @pl.when(c == 1)
    def _():
        pltpu.sync_copy(acc_sh.at[pl.ds(rowbase, RPT)],
                        a1.at[pl.ds(rowbase, RPT)])


def _tc_mm_body(x_ref, w1_ref, xw_ref):
    xw_ref[...] = jnp.dot(x_ref[...], w1_ref[...], preferred_element_type=f32)


def _tc_a_body(deg0, deg1, xw_ref, ylo_ref, yhi_ref, dinv_ref):
    deg = deg0[:, 0:1] + deg1[:, 0:1] + 1.0
    dinv = lax.rsqrt(deg)
    y = (xw_ref[...] * dinv).astype(bf16)
    ylo_ref[...] = y[:, :128]
    yhi_ref[...] = y[:, 128:]
    dinv_ref[...] = jnp.broadcast_to(dinv, (N, NW))


def _tc_b_body(alo, ahi, dinv, b1lo, b1hi, w2lo, w2hi, y2_ref):
    dv = dinv[:, 0:1]
    hlo = jnp.maximum(alo[...].astype(f32) * dv + b1lo[...], 0.0)
    hhi = jnp.maximum(ahi[...].astype(f32) * dv + b1hi[...], 0.0)
    y2 = (jnp.dot(hlo, w2lo[...], preferred_element_type=f32)
          + jnp.dot(hhi, w2hi[...], preferred_element_type=f32))
    y2_ref[...] = y2 * dv


def _tc_c_body(a0, a1, y2, dinv, b2, out_ref):
    out_ref[...] = ((a0[...] + a1[...] - y2[...]) * dinv[:, 0:1]) + b2[...]


def kernel(x, edge_index, W1, b1, W2, b2):
    ei = edge_index.astype(jnp.int32).reshape(2, EROWS, CH)
    w2p = jnp.zeros((D_HID, NW), f32).at[:, :D_OUT].set(W2)
    b1r = b1.reshape(1, D_HID)
    b2p = jnp.zeros((1, NW), f32).at[0, :D_OUT].set(b2)
    zeros_nw = jnp.zeros((N, NW), f32)
    ones_nw = jnp.ones((CH, NW), f32)

    xw = pl.pallas_call(
        _tc_mm_body,
        out_shape=jax.ShapeDtypeStruct((N, 256), f32),
    )(x, W1)

    deg0, deg1 = _deg_kernel(ei, zeros_nw, ones_nw)

    ylo, yhi, dinv = pl.pallas_call(
        _tc_a_body,
        out_shape=(jax.ShapeDtypeStruct((N, 128), bf16),
                   jax.ShapeDtypeStruct((N, 128), bf16),
                   jax.ShapeDtypeStruct((N, NW), f32)),
    )(deg0, deg1, xw)

    alo, ahi = _agg1_kernel(ylo, yhi, ei)

    y2 = pl.pallas_call(
        _tc_b_body,
        out_shape=jax.ShapeDtypeStruct((N, NW), f32),
    )(alo, ahi, dinv, b1r[:, :128], b1r[:, 128:], w2p[:128], w2p[128:])

    a0, a1 = _agg2_kernel(y2, ei)

    out_full = pl.pallas_call(
        _tc_c_body,
        out_shape=jax.ShapeDtypeStruct((N, NW), f32),
    )(a0, a1, y2, dinv, b2p)

    return out_full[:, :D_OUT]


# speedup vs baseline: 1.0095x; 1.0095x over previous
import functools

import jax
import jax.numpy as jnp
from jax import lax
from jax.experimental import pallas as pl
from jax.experimental.pallas import tpu as pltpu
from jax.experimental.pallas import tpu_sc as plsc

N = 10000
D_IN = 256
D_HID = 256
D_OUT = 4
N_EDGES = 160000

NC = 2
NS = 16
CH = 128

RPT = N // NS
EROWS = N_EDGES // CH
C1 = EROWS // NS
R1 = EROWS % NS
C2 = EROWS // (NC * NS)
R2 = EROWS % (NC * NS)
NW = 16

_mesh = plsc.VectorSubcoreMesh(
    core_axis_name="c", subcore_axis_name="s", num_cores=NC, num_subcores=NS)

_sc_params = pltpu.CompilerParams(use_tc_tiling_on_sc=False)

f32 = jnp.float32
bf16 = jnp.bfloat16


def _ring(n, wait_gather, fire_scatter, wait_scatter, fire_gather):
    fire_gather(0, 0)
    fire_gather(1, 1)

    def body(i, carry):
        for b in range(4):
            j = i * 4 + b

            @pl.when(j < n)
            def _():
                wait_gather(j, b)
                fire_scatter(j, b)

            bn = (b + 2) % 4

            @pl.when((j >= 2) & (j <= n + 1))
            def _():
                wait_scatter(j - 2, bn)

            @pl.when(j + 2 < n)
            def _():
                fire_gather(j + 2, bn)
        return carry

    lax.fori_loop(0, lax.div(n + 5, 4), body, 0)


@functools.partial(
    pl.kernel,
    out_type=(jax.ShapeDtypeStruct((N, NW), f32),
              jax.ShapeDtypeStruct((N, NW), f32)),
    mesh=_mesh,
    compiler_params=_sc_params,
    scratch_types=[pltpu.VMEM((C2 + 1, CH), jnp.int32),
                   pltpu.VMEM((CH, NW), f32),
                   pltpu.VMEM_SHARED((N, NW), f32),
                   pltpu.SemaphoreType.DMA],
)
def _deg_kernel(ei, zeros_hbm, ones_hbm, deg0, deg1, idx_v, ones_v, acc_sh,
                sem):
    c = lax.axis_index("c")
    s = lax.axis_index("s")
    rowbase = s * RPT
    pltpu.sync_copy(ones_hbm, ones_v)
    pltpu.sync_copy(zeros_hbm.at[pl.ds(rowbase, RPT)],
                    acc_sh.at[pl.ds(rowbase, RPT)])
    wid = c * NS + s
    nrow = C2 + (wid < R2)
    erow = wid * C2 + jnp.minimum(wid, R2)
    pltpu.sync_copy(ei.at[1, pl.ds(erow, C2)], idx_v.at[pl.ds(0, C2)])

    @pl.when(wid < R2)
    def _():
        pltpu.sync_copy(ei.at[1, pl.ds(erow + C2, 1)], idx_v.at[pl.ds(C2, 1)])

    plsc.subcore_barrier()

    def body(j, carry):
        pltpu.async_copy(ones_v, acc_sh.at[idx_v.at[j]], sem, add=True)
        return carry

    lax.fori_loop(0, nrow, body, 0)

    def drain(j, carry):
        pltpu.make_async_copy(ones_v, acc_sh.at[idx_v.at[j]], sem).wait()
        return carry

    lax.fori_loop(0, nrow, drain, 0)
    plsc.subcore_barrier()

    @pl.when(c == 0)
    def _():
        pltpu.sync_copy(acc_sh.at[pl.ds(rowbase, RPT)],
                        deg0.at[pl.ds(rowbase, RPT)])

    @pl.when(c == 1)
    def _():
        pltpu.sync_copy(acc_sh.at[pl.ds(rowbase, RPT)],
                        deg1.at[pl.ds(rowbase, RPT)])


@functools.partial(
    pl.kernel,
    out_type=(jax.ShapeDtypeStruct((N, 128), bf16),
              jax.ShapeDtypeStruct((N, 128), bf16)),
    mesh=_mesh,
    compiler_params=_sc_params,
    scratch_types=[pltpu.VMEM((C1 + 1, CH), jnp.int32),
                   pltpu.VMEM((C1 + 1, CH), jnp.int32),
                   pltpu.VMEM((4 * CH, 128), bf16),
                   pltpu.VMEM_SHARED((N, 128), bf16),
                   pltpu.SemaphoreType.DMA, pltpu.SemaphoreType.DMA,
                   pltpu.SemaphoreType.DMA, pltpu.SemaphoreType.DMA,
                   pltpu.SemaphoreType.DMA, pltpu.SemaphoreType.DMA,
                   pltpu.SemaphoreType.DMA, pltpu.SemaphoreType.DMA],
)
def _agg1_kernel(ylo, yhi, ei, alo, ahi,
                 src_v, dst_v, rows_v, acc_sh,
                 g0, g1, g2, g3, s0, s1, s2, s3):
    gsems = (g0, g1, g2, g3)
    ssems = (s0, s1, s2, s3)
    c = lax.axis_index("c")
    s = lax.axis_index("s")
    rowbase = s * RPT
    nrow = C1 + (s < R1)
    erow = s * C1 + jnp.minimum(s, R1)
    pltpu.sync_copy(ei.at[0, pl.ds(erow, C1)], src_v.at[pl.ds(0, C1)])
    pltpu.sync_copy(ei.at[1, pl.ds(erow, C1)], dst_v.at[pl.ds(0, C1)])

    @pl.when(s < R1)
    def _():
        pltpu.sync_copy(ei.at[0, pl.ds(erow + C1, 1)], src_v.at[pl.ds(C1, 1)])
        pltpu.sync_copy(ei.at[1, pl.ds(erow + C1, 1)], dst_v.at[pl.ds(C1, 1)])

    @pl.when(c == 0)
    def _():
        pltpu.sync_copy(ylo.at[pl.ds(rowbase, RPT)],
                        acc_sh.at[pl.ds(rowbase, RPT)])

    @pl.when(c == 1)
    def _():
        pltpu.sync_copy(yhi.at[pl.ds(rowbase, RPT)],
                        acc_sh.at[pl.ds(rowbase, RPT)])

    plsc.subcore_barrier()

    def buf(b):
        return rows_v.at[pl.ds(b * CH, CH)]

    def fire_gather(jn, b):
        @pl.when(c == 0)
        def _():
            pltpu.async_copy(ylo.at[src_v.at[jn]], buf(b), gsems[b])

        @pl.when(c == 1)
        def _():
            pltpu.async_copy(yhi.at[src_v.at[jn]], buf(b), gsems[b])

    def wait_gather(j, b):
        pltpu.make_async_copy(ylo.at[src_v.at[j]], buf(b), gsems[b]).wait()

    def fire_scatter(j, b):
        pltpu.async_copy(buf(b), acc_sh.at[dst_v.at[j]], ssems[b], add=True)

    def wait_scatter(j, b):
        pltpu.make_async_copy(buf(b), acc_sh.at[dst_v.at[j]],
                              ssems[b]).wait()

    _ring(nrow, wait_gather, fire_scatter, wait_scatter, fire_gather)
    plsc.subcore_barrier()

    @pl.when(c == 0)
    def _():
        pltpu.sync_copy(acc_sh.at[pl.ds(rowbase, RPT)],
                        alo.at[pl.ds(rowbase, RPT)])

    @pl.when(c == 1)
    def _():
        pltpu.sync_copy(acc_sh.at[pl.ds(rowbase, RPT)],
                        ahi.at[pl.ds(rowbase, RPT)])


@functools.partial(
    pl.kernel,
    out_type=(jax.ShapeDtypeStruct((N, NW), f32),
              jax.ShapeDtypeStruct((N, NW), f32)),
    mesh=_mesh,
    compiler_params=_sc_params,
    scratch_types=[pltpu.VMEM((C2 + 1, CH), jnp.int32),
                   pltpu.VMEM((C2 + 1, CH), jnp.int32),
                   pltpu.VMEM((4 * CH, NW), f32),
                   pltpu.VMEM_SHARED((N, NW), f32),
                   pltpu.SemaphoreType.DMA, pltpu.SemaphoreType.DMA,
                   pltpu.SemaphoreType.DMA, pltpu.SemaphoreType.DMA,
                   pltpu.SemaphoreType.DMA, pltpu.SemaphoreType.DMA,
                   pltpu.SemaphoreType.DMA, pltpu.SemaphoreType.DMA],
)
def _agg2_kernel(y2, ei, a0, a1, src_v, dst_v, rows_v, acc_sh,
                 g0, g1, g2, g3, s0, s1, s2, s3):
    gsems = (g0, g1, g2, g3)
    ssems = (s0, s1, s2, s3)
    c = lax.axis_index("c")
    s = lax.axis_index("s")
    rowbase = s * RPT
    wid = c * NS + s
    nrow = C2 + (wid < R2)
    erow = wid * C2 + jnp.minimum(wid, R2)
    pltpu.sync_copy(ei.at[0, pl.ds(erow, C2)], src_v.at[pl.ds(0, C2)])
    pltpu.sync_copy(ei.at[1, pl.ds(erow, C2)], dst_v.at[pl.ds(0, C2)])

    @pl.when(wid < R2)
    def _():
        pltpu.sync_copy(ei.at[0, pl.ds(erow + C2, 1)], src_v.at[pl.ds(C2, 1)])
        pltpu.sync_copy(ei.at[1, pl.ds(erow + C2, 1)], dst_v.at[pl.ds(C2, 1)])
    pltpu.sync_copy(y2.at[pl.ds(rowbase, RPT)],
                    acc_sh.at[pl.ds(rowbase, RPT)])
    plsc.subcore_barrier()

    def buf(b):
        return rows_v.at[pl.ds(b * CH, CH)]

    def fire_gather(jn, b):
        pltpu.async_copy(y2.at[src_v.at[jn]], buf(b), gsems[b])

    def wait_gather(j, b):
        pltpu.make_async_copy(y2.at[src_v.at[j]], buf(b), gsems[b]).wait()

    def fire_scatter(j, b):
        pltpu.async_copy(buf(b), acc_sh.at[dst_v.at[j]], ssems[b], add=True)

    def wait_scatter(j, b):
        pltpu.make_async_copy(buf(b), acc_sh.at[dst_v.at[j]],
                              ssems[b]).wait()

    _ring(nrow, wait_gather, fire_scatter, wait_scatter, fire_gather)
    plsc.subcore_barrier()

    @pl.when(c == 0)
    def _():
        pltpu.sync_copy(acc_sh.at[pl.ds(rowbase, RPT)],
                        a0.at[pl.ds(rowbase, RPT)])

    @pl.when(c == 1)
    def _():
        pltpu.sync_copy(acc_sh.at[pl.ds(rowbase, RPT)],
                        a1.at[pl.ds(rowbase, RPT)])


def _tc_a_body(deg0, deg1, x_ref, w1_ref, ylo_ref, yhi_ref, dinv_ref):
    deg = deg0[:, 0:1] + deg1[:, 0:1] + 1.0
    dinv = lax.rsqrt(deg)
    xw = jnp.dot(x_ref[...], w1_ref[...], preferred_element_type=f32)
    y = (xw * dinv).astype(bf16)
    ylo_ref[...] = y[:, :128]
    yhi_ref[...] = y[:, 128:]
    dinv_ref[...] = jnp.broadcast_to(dinv, (N, NW))


def _tc_b_body(alo, ahi, dinv, b1lo, b1hi, w2lo, w2hi, y2_ref):
    dv = dinv[:, 0:1]
    hlo = jnp.maximum(alo[...].astype(f32) * dv + b1lo[...], 0.0)
    hhi = jnp.maximum(ahi[...].astype(f32) * dv + b1hi[...], 0.0)
    y2 = (jnp.dot(hlo, w2lo[...], preferred_element_type=f32)
          + jnp.dot(hhi, w2hi[...], preferred_element_type=f32))
    y2_ref[...] = y2 * dv


def _tc_c_body(a0, a1, y2, dinv, b2, out_ref):
    o = ((a0[...] + a1[...] - y2[...]) * dinv[:, 0:1]) + b2[...]
    out_ref[...] = o[:, :D_OUT]


def kernel(x, edge_index, W1, b1, W2, b2):
    ei = edge_index.astype(jnp.int32).reshape(2, EROWS, CH)
    w2p = jnp.zeros((D_HID, NW), f32).at[:, :D_OUT].set(W2)
    b1r = b1.reshape(1, D_HID)
    b2p = jnp.zeros((1, NW), f32).at[0, :D_OUT].set(b2)
    zeros_nw = jnp.zeros((N, NW), f32)
    ones_nw = jnp.ones((CH, NW), f32)

    deg0, deg1 = _deg_kernel(ei, zeros_nw, ones_nw)

    ylo, yhi, dinv = pl.pallas_call(
        _tc_a_body,
        out_shape=(jax.ShapeDtypeStruct((N, 128), bf16),
                   jax.ShapeDtypeStruct((N, 128), bf16),
                   jax.ShapeDtypeStruct((N, NW), f32)),
    )(deg0, deg1, x, W1)

    alo, ahi = _agg1_kernel(ylo, yhi, ei)

    y2 = pl.pallas_call(
        _tc_b_body,
        out_shape=jax.ShapeDtypeStruct((N, NW), f32),
    )(alo, ahi, dinv, b1r[:, :128], b1r[:, 128:], w2p[:128], w2p[128:])

    a0, a1 = _agg2_kernel(y2, ei)

    return pl.pallas_call(
        _tc_c_body,
        out_shape=jax.ShapeDtypeStruct((N, D_OUT), f32),
    )(a0, a1, y2, dinv, b2p)


# speedup vs baseline: 1.0461x; 1.0362x over previous
import functools

import jax
import jax.numpy as jnp
from jax import lax
from jax.experimental import pallas as pl
from jax.experimental.pallas import tpu as pltpu
from jax.experimental.pallas import tpu_sc as plsc

N = 10000
D_IN = 256
D_HID = 256
D_OUT = 4
N_EDGES = 160000

NC = 2
NS = 16
CH = 128

RPT = N // NS
EROWS = N_EDGES // CH
C1 = EROWS // NS
R1 = EROWS % NS
C2 = EROWS // (NC * NS)
R2 = EROWS % (NC * NS)
NW = 16

_mesh = plsc.VectorSubcoreMesh(
    core_axis_name="c", subcore_axis_name="s", num_cores=NC, num_subcores=NS)

_sc_params = pltpu.CompilerParams(use_tc_tiling_on_sc=False)

f32 = jnp.float32
bf16 = jnp.bfloat16


def _ring(n, wait_gather, fire_scatter, wait_scatter, fire_gather,
          nbuf=4):
    ahead = nbuf // 2
    for b in range(ahead):
        fire_gather(b, b)

    def body(i, carry):
        for b in range(nbuf):
            j = i * nbuf + b

            @pl.when(j < n)
            def _():
                wait_gather(j, b)
                fire_scatter(j, b)

            bn = (b + ahead) % nbuf

            @pl.when((j >= ahead) & (j <= n + ahead - 1))
            def _():
                wait_scatter(j - ahead, bn)

            @pl.when(j + ahead < n)
            def _():
                fire_gather(j + ahead, bn)
        return carry

    lax.fori_loop(0, lax.div(n + 2 * ahead + nbuf - 1, nbuf), body, 0)


@functools.partial(
    pl.kernel,
    out_type=(jax.ShapeDtypeStruct((N, NW), f32),
              jax.ShapeDtypeStruct((N, NW), f32)),
    mesh=_mesh,
    compiler_params=_sc_params,
    scratch_types=[pltpu.VMEM((C2 + 1, CH), jnp.int32),
                   pltpu.VMEM((CH, NW), f32),
                   pltpu.VMEM_SHARED((N, NW), f32),
                   pltpu.SemaphoreType.DMA],
)
def _deg_kernel(ei, zeros_hbm, ones_hbm, deg0, deg1, idx_v, ones_v, acc_sh,
                sem):
    c = lax.axis_index("c")
    s = lax.axis_index("s")
    rowbase = s * RPT
    pltpu.sync_copy(ones_hbm, ones_v)
    pltpu.sync_copy(zeros_hbm.at[pl.ds(rowbase, RPT)],
                    acc_sh.at[pl.ds(rowbase, RPT)])
    wid = c * NS + s
    nrow = C2 + (wid < R2)
    erow = wid * C2 + jnp.minimum(wid, R2)
    pltpu.sync_copy(ei.at[1, pl.ds(erow, C2)], idx_v.at[pl.ds(0, C2)])

    @pl.when(wid < R2)
    def _():
        pltpu.sync_copy(ei.at[1, pl.ds(erow + C2, 1)], idx_v.at[pl.ds(C2, 1)])

    plsc.subcore_barrier()

    def body(j, carry):
        pltpu.async_copy(ones_v, acc_sh.at[idx_v.at[j]], sem, add=True)
        return carry

    lax.fori_loop(0, nrow, body, 0)

    def drain(j, carry):
        pltpu.make_async_copy(ones_v, acc_sh.at[idx_v.at[j]], sem).wait()
        return carry

    lax.fori_loop(0, nrow, drain, 0)
    plsc.subcore_barrier()

    @pl.when(c == 0)
    def _():
        pltpu.sync_copy(acc_sh.at[pl.ds(rowbase, RPT)],
                        deg0.at[pl.ds(rowbase, RPT)])

    @pl.when(c == 1)
    def _():
        pltpu.sync_copy(acc_sh.at[pl.ds(rowbase, RPT)],
                        deg1.at[pl.ds(rowbase, RPT)])


@functools.partial(
    pl.kernel,
    out_type=(jax.ShapeDtypeStruct((N, 128), bf16),
              jax.ShapeDtypeStruct((N, 128), bf16)),
    mesh=_mesh,
    compiler_params=_sc_params,
    scratch_types=[pltpu.VMEM((C1 + 1, CH), jnp.int32),
                   pltpu.VMEM((C1 + 1, CH), jnp.int32),
                   pltpu.VMEM((8 * CH, 128), bf16),
                   pltpu.VMEM_SHARED((N, 128), bf16),
                   pltpu.SemaphoreType.DMA, pltpu.SemaphoreType.DMA,
                   pltpu.SemaphoreType.DMA, pltpu.SemaphoreType.DMA,
                   pltpu.SemaphoreType.DMA, pltpu.SemaphoreType.DMA,
                   pltpu.SemaphoreType.DMA, pltpu.SemaphoreType.DMA,
                   pltpu.SemaphoreType.DMA, pltpu.SemaphoreType.DMA,
                   pltpu.SemaphoreType.DMA, pltpu.SemaphoreType.DMA,
                   pltpu.SemaphoreType.DMA, pltpu.SemaphoreType.DMA,
                   pltpu.SemaphoreType.DMA, pltpu.SemaphoreType.DMA],
)
def _agg1_kernel(ylo, yhi, ei, alo, ahi,
                 src_v, dst_v, rows_v, acc_sh,
                 g0, g1, g2, g3, g4, g5, g6, g7,
                 s0, s1, s2, s3, s4, s5, s6, s7):
    gsems = (g0, g1, g2, g3, g4, g5, g6, g7)
    ssems = (s0, s1, s2, s3, s4, s5, s6, s7)
    c = lax.axis_index("c")
    s = lax.axis_index("s")
    rowbase = s * RPT
    nrow = C1 + (s < R1)
    erow = s * C1 + jnp.minimum(s, R1)
    pltpu.sync_copy(ei.at[0, pl.ds(erow, C1)], src_v.at[pl.ds(0, C1)])
    pltpu.sync_copy(ei.at[1, pl.ds(erow, C1)], dst_v.at[pl.ds(0, C1)])

    @pl.when(s < R1)
    def _():
        pltpu.sync_copy(ei.at[0, pl.ds(erow + C1, 1)], src_v.at[pl.ds(C1, 1)])
        pltpu.sync_copy(ei.at[1, pl.ds(erow + C1, 1)], dst_v.at[pl.ds(C1, 1)])

    @pl.when(c == 0)
    def _():
        pltpu.sync_copy(ylo.at[pl.ds(rowbase, RPT)],
                        acc_sh.at[pl.ds(rowbase, RPT)])

    @pl.when(c == 1)
    def _():
        pltpu.sync_copy(yhi.at[pl.ds(rowbase, RPT)],
                        acc_sh.at[pl.ds(rowbase, RPT)])

    plsc.subcore_barrier()

    def buf(b):
        return rows_v.at[pl.ds(b * CH, CH)]

    def fire_gather(jn, b):
        @pl.when(c == 0)
        def _():
            pltpu.async_copy(ylo.at[src_v.at[jn]], buf(b), gsems[b])

        @pl.when(c == 1)
        def _():
            pltpu.async_copy(yhi.at[src_v.at[jn]], buf(b), gsems[b])

    def wait_gather(j, b):
        pltpu.make_async_copy(ylo.at[src_v.at[j]], buf(b), gsems[b]).wait()

    def fire_scatter(j, b):
        pltpu.async_copy(buf(b), acc_sh.at[dst_v.at[j]], ssems[b], add=True)

    def wait_scatter(j, b):
        pltpu.make_async_copy(buf(b), acc_sh.at[dst_v.at[j]],
                              ssems[b]).wait()

    _ring(nrow, wait_gather, fire_scatter, wait_scatter, fire_gather,
          nbuf=8)
    plsc.subcore_barrier()

    @pl.when(c == 0)
    def _():
        pltpu.sync_copy(acc_sh.at[pl.ds(rowbase, RPT)],
                        alo.at[pl.ds(rowbase, RPT)])

    @pl.when(c == 1)
    def _():
        pltpu.sync_copy(acc_sh.at[pl.ds(rowbase, RPT)],
                        ahi.at[pl.ds(rowbase, RPT)])


@functools.partial(
    pl.kernel,
    out_type=(jax.ShapeDtypeStruct((N, NW), f32),
              jax.ShapeDtypeStruct((N, NW), f32)),
    mesh=_mesh,
    compiler_params=_sc_params,
    scratch_types=[pltpu.VMEM((C2 + 1, CH), jnp.int32),
                   pltpu.VMEM((C2 + 1, CH), jnp.int32),
                   pltpu.VMEM((4 * CH, NW), f32),
                   pltpu.VMEM_SHARED((N, NW), f32),
                   pltpu.SemaphoreType.DMA, pltpu.SemaphoreType.DMA,
                   pltpu.SemaphoreType.DMA, pltpu.SemaphoreType.DMA,
                   pltpu.SemaphoreType.DMA, pltpu.SemaphoreType.DMA,
                   pltpu.SemaphoreType.DMA, pltpu.SemaphoreType.DMA],
)
def _agg2_kernel(y2, ei, a0, a1, src_v, dst_v, rows_v, acc_sh,
                 g0, g1, g2, g3, s0, s1, s2, s3):
    gsems = (g0, g1, g2, g3)
    ssems = (s0, s1, s2, s3)
    c = lax.axis_index("c")
    s = lax.axis_index("s")
    rowbase = s * RPT
    wid = c * NS + s
    nrow = C2 + (wid < R2)
    erow = wid * C2 + jnp.minimum(wid, R2)
    pltpu.sync_copy(ei.at[0, pl.ds(erow, C2)], src_v.at[pl.ds(0, C2)])
    pltpu.sync_copy(ei.at[1, pl.ds(erow, C2)], dst_v.at[pl.ds(0, C2)])

    @pl.when(wid < R2)
    def _():
        pltpu.sync_copy(ei.at[0, pl.ds(erow + C2, 1)], src_v.at[pl.ds(C2, 1)])
        pltpu.sync_copy(ei.at[1, pl.ds(erow + C2, 1)], dst_v.at[pl.ds(C2, 1)])
    pltpu.sync_copy(y2.at[pl.ds(rowbase, RPT)],
                    acc_sh.at[pl.ds(rowbase, RPT)])
    plsc.subcore_barrier()

    def buf(b):
        return rows_v.at[pl.ds(b * CH, CH)]

    def fire_gather(jn, b):
        pltpu.async_copy(y2.at[src_v.at[jn]], buf(b), gsems[b])

    def wait_gather(j, b):
        pltpu.make_async_copy(y2.at[src_v.at[j]], buf(b), gsems[b]).wait()

    def fire_scatter(j, b):
        pltpu.async_copy(buf(b), acc_sh.at[dst_v.at[j]], ssems[b], add=True)

    def wait_scatter(j, b):
        pltpu.make_async_copy(buf(b), acc_sh.at[dst_v.at[j]],
                              ssems[b]).wait()

    _ring(nrow, wait_gather, fire_scatter, wait_scatter, fire_gather)
    plsc.subcore_barrier()

    @pl.when(c == 0)
    def _():
        pltpu.sync_copy(acc_sh.at[pl.ds(rowbase, RPT)],
                        a0.at[pl.ds(rowbase, RPT)])

    @pl.when(c == 1)
    def _():
        pltpu.sync_copy(acc_sh.at[pl.ds(rowbase, RPT)],
                        a1.at[pl.ds(rowbase, RPT)])


def _tc_a_body(deg0, deg1, x_ref, w1_ref, ylo_ref, yhi_ref, dinv_ref):
    deg = deg0[:, 0:1] + deg1[:, 0:1] + 1.0
    dinv = lax.rsqrt(deg)
    xw = jnp.dot(x_ref[...], w1_ref[...], preferred_element_type=f32)
    y = (xw * dinv).astype(bf16)
    ylo_ref[...] = y[:, :128]
    yhi_ref[...] = y[:, 128:]
    dinv_ref[...] = jnp.broadcast_to(dinv, (N, NW))


def _tc_b_body(alo, ahi, dinv, b1lo, b1hi, w2lo, w2hi, y2_ref):
    dv = dinv[:, 0:1]
    hlo = jnp.maximum(alo[...].astype(f32) * dv + b1lo[...], 0.0)
    hhi = jnp.maximum(ahi[...].astype(f32) * dv + b1hi[...], 0.0)
    y2 = (jnp.dot(hlo, w2lo[...], preferred_element_type=f32)
          + jnp.dot(hhi, w2hi[...], preferred_element_type=f32))
    y2_ref[...] = y2 * dv


def _tc_c_body(a0, a1, y2, dinv, b2, out_ref):
    o = ((a0[...] + a1[...] - y2[...]) * dinv[:, 0:1]) + b2[...]
    out_ref[...] = o[:, :D_OUT]


def kernel(x, edge_index, W1, b1, W2, b2):
    ei = edge_index.astype(jnp.int32).reshape(2, EROWS, CH)
    w2p = jnp.zeros((D_HID, NW), f32).at[:, :D_OUT].set(W2)
    b1r = b1.reshape(1, D_HID)
    b2p = jnp.zeros((1, NW), f32).at[0, :D_OUT].set(b2)
    zeros_nw = jnp.zeros((N, NW), f32)
    ones_nw = jnp.ones((CH, NW), f32)

    deg0, deg1 = _deg_kernel(ei, zeros_nw, ones_nw)

    ylo, yhi, dinv = pl.pallas_call(
        _tc_a_body,
        out_shape=(jax.ShapeDtypeStruct((N, 128), bf16),
                   jax.ShapeDtypeStruct((N, 128), bf16),
                   jax.ShapeDtypeStruct((N, NW), f32)),
    )(deg0, deg1, x, W1)

    alo, ahi = _agg1_kernel(ylo, yhi, ei)

    y2 = pl.pallas_call(
        _tc_b_body,
        out_shape=jax.ShapeDtypeStruct((N, NW), f32),
    )(alo, ahi, dinv, b1r[:, :128], b1r[:, 128:], w2p[:128], w2p[128:])

    a0, a1 = _agg2_kernel(y2, ei)

    return pl.pallas_call(
        _tc_c_body,
        out_shape=jax.ShapeDtypeStruct((N, D_OUT), f32),
    )(a0, a1, y2, dinv, b2p)


# speedup vs baseline: 1.0819x; 1.0342x over previous
import functools

import jax
import jax.numpy as jnp
from jax import lax
from jax.experimental import pallas as pl
from jax.experimental.pallas import tpu as pltpu
from jax.experimental.pallas import tpu_sc as plsc

N = 10000
D_IN = 256
D_HID = 256
D_OUT = 4
N_EDGES = 160000

NC = 2
NS = 16
CH = 128

RPT = N // NS
EROWS = N_EDGES // CH
C1 = EROWS // NS
R1 = EROWS % NS
C2 = EROWS // (NC * NS)
R2 = EROWS % (NC * NS)
NW = 16

_mesh = plsc.VectorSubcoreMesh(
    core_axis_name="c", subcore_axis_name="s", num_cores=NC, num_subcores=NS)

_sc_params = pltpu.CompilerParams(use_tc_tiling_on_sc=False)

f32 = jnp.float32
bf16 = jnp.bfloat16


def _ring(n, wait_gather, fire_scatter, wait_scatter, fire_gather,
          nbuf=4):
    ahead = nbuf // 2
    for b in range(ahead):
        fire_gather(b, b)

    def body(i, carry):
        for b in range(nbuf):
            j = i * nbuf + b

            @pl.when(j < n)
            def _():
                wait_gather(j, b)
                fire_scatter(j, b)

            bn = (b + ahead) % nbuf

            @pl.when((j >= ahead) & (j <= n + ahead - 1))
            def _():
                wait_scatter(j - ahead, bn)

            @pl.when(j + ahead < n)
            def _():
                fire_gather(j + ahead, bn)
        return carry

    lax.fori_loop(0, lax.div(n + 2 * ahead + nbuf - 1, nbuf), body, 0)


@functools.partial(
    pl.kernel,
    out_type=(jax.ShapeDtypeStruct((N, NW), f32),
              jax.ShapeDtypeStruct((N, NW), f32)),
    mesh=_mesh,
    compiler_params=_sc_params,
    scratch_types=[pltpu.VMEM((C2 + 1, CH), jnp.int32),
                   pltpu.VMEM((CH, NW), f32),
                   pltpu.VMEM_SHARED((N, NW), f32),
                   pltpu.SemaphoreType.DMA],
)
def _deg_kernel(ei, zeros_hbm, ones_hbm, deg0, deg1, idx_v, ones_v, acc_sh,
                sem):
    c = lax.axis_index("c")
    s = lax.axis_index("s")
    rowbase = s * RPT
    pltpu.sync_copy(ones_hbm, ones_v)
    pltpu.sync_copy(zeros_hbm.at[pl.ds(rowbase, RPT)],
                    acc_sh.at[pl.ds(rowbase, RPT)])
    wid = c * NS + s
    nrow = C2 + (wid < R2)
    erow = wid * C2 + jnp.minimum(wid, R2)
    pltpu.sync_copy(ei.at[1, pl.ds(erow, C2)], idx_v.at[pl.ds(0, C2)])

    @pl.when(wid < R2)
    def _():
        pltpu.sync_copy(ei.at[1, pl.ds(erow + C2, 1)], idx_v.at[pl.ds(C2, 1)])

    plsc.subcore_barrier()

    def body(j, carry):
        pltpu.async_copy(ones_v, acc_sh.at[idx_v.at[j]], sem, add=True)
        return carry

    lax.fori_loop(0, nrow, body, 0)

    def drain(j, carry):
        pltpu.make_async_copy(ones_v, acc_sh.at[idx_v.at[j]], sem).wait()
        return carry

    lax.fori_loop(0, nrow, drain, 0)
    plsc.subcore_barrier()

    @pl.when(c == 0)
    def _():
        pltpu.sync_copy(acc_sh.at[pl.ds(rowbase, RPT)],
                        deg0.at[pl.ds(rowbase, RPT)])

    @pl.when(c == 1)
    def _():
        pltpu.sync_copy(acc_sh.at[pl.ds(rowbase, RPT)],
                        deg1.at[pl.ds(rowbase, RPT)])


@functools.partial(
    pl.kernel,
    out_type=(jax.ShapeDtypeStruct((N, 128), bf16),
              jax.ShapeDtypeStruct((N, 128), bf16)),
    mesh=_mesh,
    compiler_params=_sc_params,
    scratch_types=[pltpu.VMEM((C1 + 1, CH), jnp.int32),
                   pltpu.VMEM((C1 + 1, CH), jnp.int32),
                   pltpu.VMEM((8 * CH, 128), bf16),
                   pltpu.VMEM_SHARED((N, 128), bf16),
                   pltpu.SemaphoreType.DMA, pltpu.SemaphoreType.DMA,
                   pltpu.SemaphoreType.DMA, pltpu.SemaphoreType.DMA,
                   pltpu.SemaphoreType.DMA, pltpu.SemaphoreType.DMA,
                   pltpu.SemaphoreType.DMA, pltpu.SemaphoreType.DMA,
                   pltpu.SemaphoreType.DMA, pltpu.SemaphoreType.DMA,
                   pltpu.SemaphoreType.DMA, pltpu.SemaphoreType.DMA,
                   pltpu.SemaphoreType.DMA, pltpu.SemaphoreType.DMA,
                   pltpu.SemaphoreType.DMA, pltpu.SemaphoreType.DMA],
)
def _agg1_kernel(ylo, yhi, ei, alo, ahi,
                 src_v, dst_v, rows_v, acc_sh,
                 g0, g1, g2, g3, g4, g5, g6, g7,
                 s0, s1, s2, s3, s4, s5, s6, s7):
    gsems = (g0, g1, g2, g3, g4, g5, g6, g7)
    ssems = (s0, s1, s2, s3, s4, s5, s6, s7)
    c = lax.axis_index("c")
    s = lax.axis_index("s")
    rowbase = s * RPT
    nrow = C1 + (s < R1)
    erow = s * C1 + jnp.minimum(s, R1)
    pltpu.sync_copy(ei.at[0, pl.ds(erow, C1)], src_v.at[pl.ds(0, C1)])
    pltpu.sync_copy(ei.at[1, pl.ds(erow, C1)], dst_v.at[pl.ds(0, C1)])

    @pl.when(s < R1)
    def _():
        pltpu.sync_copy(ei.at[0, pl.ds(erow + C1, 1)], src_v.at[pl.ds(C1, 1)])
        pltpu.sync_copy(ei.at[1, pl.ds(erow + C1, 1)], dst_v.at[pl.ds(C1, 1)])

    @pl.when(c == 0)
    def _():
        pltpu.sync_copy(ylo.at[pl.ds(rowbase, RPT)],
                        acc_sh.at[pl.ds(rowbase, RPT)])

    @pl.when(c == 1)
    def _():
        pltpu.sync_copy(yhi.at[pl.ds(rowbase, RPT)],
                        acc_sh.at[pl.ds(rowbase, RPT)])

    plsc.subcore_barrier()

    def buf(b):
        return rows_v.at[pl.ds(b * CH, CH)]

    def fire_gather(jn, b):
        @pl.when(c == 0)
        def _():
            pltpu.async_copy(ylo.at[src_v.at[jn]], buf(b), gsems[b])

        @pl.when(c == 1)
        def _():
            pltpu.async_copy(yhi.at[src_v.at[jn]], buf(b), gsems[b])

    def wait_gather(j, b):
        pltpu.make_async_copy(ylo.at[src_v.at[j]], buf(b), gsems[b]).wait()

    def fire_scatter(j, b):
        pltpu.async_copy(buf(b), acc_sh.at[dst_v.at[j]], ssems[b], add=True)

    def wait_scatter(j, b):
        pltpu.make_async_copy(buf(b), acc_sh.at[dst_v.at[j]],
                              ssems[b]).wait()

    _ring(nrow, wait_gather, fire_scatter, wait_scatter, fire_gather,
          nbuf=8)
    plsc.subcore_barrier()

    @pl.when(c == 0)
    def _():
        pltpu.sync_copy(acc_sh.at[pl.ds(rowbase, RPT)],
                        alo.at[pl.ds(rowbase, RPT)])

    @pl.when(c == 1)
    def _():
        pltpu.sync_copy(acc_sh.at[pl.ds(rowbase, RPT)],
                        ahi.at[pl.ds(rowbase, RPT)])


@functools.partial(
    pl.kernel,
    out_type=(jax.ShapeDtypeStruct((N, NW), f32),
              jax.ShapeDtypeStruct((N, NW), f32)),
    mesh=_mesh,
    compiler_params=_sc_params,
    scratch_types=[pltpu.VMEM((C2 + 1, CH), jnp.int32),
                   pltpu.VMEM((C2 + 1, CH), jnp.int32),
                   pltpu.VMEM((8 * CH, NW), f32),
                   pltpu.VMEM_SHARED((N, NW), f32),
                   pltpu.SemaphoreType.DMA, pltpu.SemaphoreType.DMA,
                   pltpu.SemaphoreType.DMA, pltpu.SemaphoreType.DMA,
                   pltpu.SemaphoreType.DMA, pltpu.SemaphoreType.DMA,
                   pltpu.SemaphoreType.DMA, pltpu.SemaphoreType.DMA,
                   pltpu.SemaphoreType.DMA, pltpu.SemaphoreType.DMA,
                   pltpu.SemaphoreType.DMA, pltpu.SemaphoreType.DMA,
                   pltpu.SemaphoreType.DMA, pltpu.SemaphoreType.DMA,
                   pltpu.SemaphoreType.DMA, pltpu.SemaphoreType.DMA],
)
def _agg2_kernel(y2, ei, a0, a1, src_v, dst_v, rows_v, acc_sh,
                 g0, g1, g2, g3, g4, g5, g6, g7,
                 s0, s1, s2, s3, s4, s5, s6, s7):
    gsems = (g0, g1, g2, g3, g4, g5, g6, g7)
    ssems = (s0, s1, s2, s3, s4, s5, s6, s7)
    c = lax.axis_index("c")
    s = lax.axis_index("s")
    rowbase = s * RPT
    wid = c * NS + s
    nrow = C2 + (wid < R2)
    erow = wid * C2 + jnp.minimum(wid, R2)
    pltpu.sync_copy(ei.at[0, pl.ds(erow, C2)], src_v.at[pl.ds(0, C2)])
    pltpu.sync_copy(ei.at[1, pl.ds(erow, C2)], dst_v.at[pl.ds(0, C2)])

    @pl.when(wid < R2)
    def _():
        pltpu.sync_copy(ei.at[0, pl.ds(erow + C2, 1)], src_v.at[pl.ds(C2, 1)])
        pltpu.sync_copy(ei.at[1, pl.ds(erow + C2, 1)], dst_v.at[pl.ds(C2, 1)])
    pltpu.sync_copy(y2.at[pl.ds(rowbase, RPT)],
                    acc_sh.at[pl.ds(rowbase, RPT)])
    plsc.subcore_barrier()

    def buf(b):
        return rows_v.at[pl.ds(b * CH, CH)]

    def fire_gather(jn, b):
        pltpu.async_copy(y2.at[src_v.at[jn]], buf(b), gsems[b])

    def wait_gather(j, b):
        pltpu.make_async_copy(y2.at[src_v.at[j]], buf(b), gsems[b]).wait()

    def fire_scatter(j, b):
        pltpu.async_copy(buf(b), acc_sh.at[dst_v.at[j]], ssems[b], add=True)

    def wait_scatter(j, b):
        pltpu.make_async_copy(buf(b), acc_sh.at[dst_v.at[j]],
                              ssems[b]).wait()

    _ring(nrow, wait_gather, fire_scatter, wait_scatter, fire_gather,
          nbuf=8)
    plsc.subcore_barrier()

    @pl.when(c == 0)
    def _():
        pltpu.sync_copy(acc_sh.at[pl.ds(rowbase, RPT)],
                        a0.at[pl.ds(rowbase, RPT)])

    @pl.when(c == 1)
    def _():
        pltpu.sync_copy(acc_sh.at[pl.ds(rowbase, RPT)],
                        a1.at[pl.ds(rowbase, RPT)])


def _tc_a_body(deg0, deg1, x_ref, w1_ref, ylo_ref, yhi_ref, dinv_ref):
    deg = deg0[:, 0:1] + deg1[:, 0:1] + 1.0
    dinv = lax.rsqrt(deg)
    xw = jnp.dot(x_ref[...], w1_ref[...], preferred_element_type=f32)
    y = (xw * dinv).astype(bf16)
    ylo_ref[...] = y[:, :128]
    yhi_ref[...] = y[:, 128:]
    dinv_ref[...] = jnp.broadcast_to(dinv, (N, NW))


def _tc_b_body(alo, ahi, dinv, b1lo, b1hi, w2lo, w2hi, y2_ref):
    dv = dinv[:, 0:1]
    hlo = jnp.maximum(alo[...].astype(f32) * dv + b1lo[...], 0.0)
    hhi = jnp.maximum(ahi[...].astype(f32) * dv + b1hi[...], 0.0)
    y2 = (jnp.dot(hlo, w2lo[...], preferred_element_type=f32)
          + jnp.dot(hhi, w2hi[...], preferred_element_type=f32))
    y2_ref[...] = y2 * dv


def _tc_c_body(a0, a1, y2, dinv, b2, out_ref):
    o = ((a0[...] + a1[...] - y2[...]) * dinv[:, 0:1]) + b2[...]
    out_ref[...] = o[:, :D_OUT]


def kernel(x, edge_index, W1, b1, W2, b2):
    ei = edge_index.astype(jnp.int32).reshape(2, EROWS, CH)
    w2p = jnp.zeros((D_HID, NW), f32).at[:, :D_OUT].set(W2)
    b1r = b1.reshape(1, D_HID)
    b2p = jnp.zeros((1, NW), f32).at[0, :D_OUT].set(b2)
    zeros_nw = jnp.zeros((N, NW), f32)
    ones_nw = jnp.ones((CH, NW), f32)

    deg0, deg1 = _deg_kernel(ei, zeros_nw, ones_nw)

    ylo, yhi, dinv = pl.pallas_call(
        _tc_a_body,
        out_shape=(jax.ShapeDtypeStruct((N, 128), bf16),
                   jax.ShapeDtypeStruct((N, 128), bf16),
                   jax.ShapeDtypeStruct((N, NW), f32)),
    )(deg0, deg1, x, W1)

    alo, ahi = _agg1_kernel(ylo, yhi, ei)

    y2 = pl.pallas_call(
        _tc_b_body,
        out_shape=jax.ShapeDtypeStruct((N, NW), f32),
    )(alo, ahi, dinv, b1r[:, :128], b1r[:, 128:], w2p[:128], w2p[128:])

    a0, a1 = _agg2_kernel(y2, ei)

    return pl.pallas_call(
        _tc_c_body,
        out_shape=jax.ShapeDtypeStruct((N, D_OUT), f32),
    )(a0, a1, y2, dinv, b2p)
